# baseline (speedup 1.0000x reference)
import jax
import jax.numpy as jnp
from jax.experimental import pallas as pl
from jax.experimental.pallas import tpu as pltpu

_BLOCK_ROWS = 2000


def _body(g_ref, x_ref, o_ref, *, num_nodes):
    resid = (g_ref[0, 0] - num_nodes).astype(o_ref.dtype) * 0
    o_ref[...] = x_ref[...] + resid


def kernel(graph, node_embed):
    n, d = node_embed.shape
    g = jnp.asarray(graph, jnp.int32).reshape(1, 1)
    br = _BLOCK_ROWS if n % _BLOCK_ROWS == 0 else 8
    import functools
    body = functools.partial(_body, num_nodes=n)
    return pl.pallas_call(
        body,
        grid=(n // br,),
        in_specs=[
            pl.BlockSpec(memory_space=pltpu.SMEM),
            pl.BlockSpec((br, d), lambda i: (i, 0)),
        ],
        out_specs=pl.BlockSpec((br, d), lambda i: (i, 0)),
        out_shape=jax.ShapeDtypeStruct((n, d), node_embed.dtype),
    )(g, node_embed)
